# baseline (speedup 1.0000x reference)
import functools

import numpy as np
import jax
import jax.numpy as jnp
from jax import lax
from jax.experimental import pallas as pl
from jax.experimental.pallas import tpu as pltpu

_THRESHOLD = 128
_GROUPS = 2


def _dft_mats(h: int, w: int):
    hh = np.arange(h)
    ah = np.exp(2j * np.pi * np.outer(hh, hh) / h) / h
    ww = np.arange(w)
    bw = np.exp(-2j * np.pi * np.outer(ww, ww) / w)
    aa = np.zeros((2 * h, h), np.float64)
    bb = np.zeros((2 * h, h), np.float64)
    aa[0::2] = ah.real
    aa[1::2] = ah.imag
    bb[0::2] = -ah.imag
    bb[1::2] = ah.real
    half = np.zeros((2 * h, 128), np.float32)
    half[0::2] = 0.5
    return (
        jnp.asarray(aa, jnp.float32),
        jnp.asarray(bb, jnp.float32),
        jnp.asarray(bw.real, jnp.float32),
        jnp.asarray(bw.imag, jnp.float32),
        jnp.asarray(half, jnp.float32),
    )


def _kernel(xr_ref, xi_ref, aa_ref, bb_ref, br_ref, bi_ref, half_ref, o_ref):
    g0r = xr_ref[0, 0]
    g0i = xi_ref[0, 0]
    g1r = xr_ref[0, 1]
    g1i = xi_ref[0, 1]

    e = jnp.sum(
        jnp.abs(jnp.abs(g0r) - jnp.abs(g1i)) + jnp.abs(jnp.abs(g1r) - jnp.abs(g0i)),
        axis=0,
    )

    w = e.shape[0]
    ecol = e[:, None]
    erow = e[None, :]
    lt = ecol < erow
    eq = ecol == erow
    iw = lax.broadcasted_iota(jnp.int32, (w, w), 0)
    jw = lax.broadcasted_iota(jnp.int32, (w, w), 1)
    before = iw < jw
    rank = jnp.sum(
        jnp.where(lt | (eq & before), jnp.int32(1), jnp.int32(0)), axis=0
    )

    jj = lax.broadcasted_iota(jnp.int32, (_THRESHOLD, w), 0)
    p = jnp.where(rank[None, :] == jj, jnp.float32(1.0), jnp.float32(0.0))

    mm = lambda a, b: jnp.dot(a, b, preferred_element_type=jnp.float32)

    aa = aa_ref[...]
    bb = bb_ref[...]
    bsel_r = mm(p, br_ref[...])
    bsel_i = mm(p, bi_ref[...])
    bs2 = jnp.concatenate([bsel_r, bsel_i], axis=0)
    half = half_ref[:, 0:1]

    compact = lambda m: lax.dot_general(
        m, p, (((1,), (1,)), ((), ())), preferred_element_type=jnp.float32
    )

    of_ref = o_ref
    for c, (gr, gi) in enumerate(((g0r, g0i), (g1r, g1i))):
        gsr = compact(gr)
        gsi = compact(gi)
        gp = jnp.concatenate([gsr, -gsi], axis=1)
        gq = jnp.concatenate([gsi, gsr], axis=1)
        t2 = mm(aa, gp) + mm(bb, gq)
        of_ref[0, c] = mm(t2, bs2) + half


@functools.partial(jax.jit, static_argnums=())
def kernel(Inp_AD_C_real, Inp_AD_C_imag):
    b, c, h, w = Inp_AD_C_real.shape
    aa, bb, br, bi, half = _dft_mats(h, w)
    cg = c // _GROUPS

    x_spec = pl.BlockSpec((1, cg, h, w), lambda ib, ig: (ib, ig, 0, 0))
    a_spec = pl.BlockSpec((2 * h, h), lambda ib, ig: (0, 0))
    m_spec = pl.BlockSpec((w, w), lambda ib, ig: (0, 0))
    h_spec = pl.BlockSpec((2 * h, 128), lambda ib, ig: (0, 0))
    o_spec = pl.BlockSpec((1, cg, 2 * h, w), lambda ib, ig: (ib, ig, 0, 0))
    out = pl.pallas_call(
        _kernel,
        grid=(b, _GROUPS),
        in_specs=[x_spec, x_spec, a_spec, a_spec, m_spec, m_spec, h_spec],
        out_specs=o_spec,
        out_shape=jax.ShapeDtypeStruct((b, c, 2 * h, w), jnp.float32),
        compiler_params=pltpu.CompilerParams(
            dimension_semantics=("parallel", "parallel"),
        ),
    )(Inp_AD_C_real, Inp_AD_C_imag, aa, bb, br, bi, half)
    o5 = out.reshape(b, c, h, 2, w)
    return lax.complex(o5[:, :, :, 0, :], o5[:, :, :, 1, :])


# speedup vs baseline: 1.3758x; 1.3758x over previous
import functools

import numpy as np
import jax
import jax.numpy as jnp
from jax import lax
from jax.experimental import pallas as pl
from jax.experimental.pallas import tpu as pltpu

_THRESHOLD = 128
_GROUPS = 2


def _dft_mats(h: int, w: int):
    hh = np.arange(h)
    ah = np.exp(2j * np.pi * np.outer(hh, hh) / h) / h
    ww = np.arange(w)
    bw = np.exp(-2j * np.pi * np.outer(ww, ww) / w)
    return (
        jnp.asarray(ah.real, jnp.float32),
        jnp.asarray(ah.imag, jnp.float32),
        jnp.asarray(ah.real + ah.imag, jnp.float32),
        jnp.asarray(bw.real, jnp.float32),
        jnp.asarray(bw.imag, jnp.float32),
    )


def _kernel(xr_ref, xi_ref, ar_ref, ai_ref, as_ref, br_ref, bi_ref, or_ref, oi_ref):
    g0r = xr_ref[0, 0]
    g0i = xi_ref[0, 0]
    g1r = xr_ref[0, 1]
    g1i = xi_ref[0, 1]

    e = jnp.sum(
        jnp.abs(jnp.abs(g0r) - jnp.abs(g1i)) + jnp.abs(jnp.abs(g1r) - jnp.abs(g0i)),
        axis=0,
    )

    w = e.shape[0]
    ecol = e[:, None]
    erow = e[None, :]
    lt = ecol < erow
    eq = ecol == erow
    iw = lax.broadcasted_iota(jnp.int32, (w, w), 0)
    jw = lax.broadcasted_iota(jnp.int32, (w, w), 1)
    before = iw < jw
    rank = jnp.sum(
        jnp.where(lt | (eq & before), jnp.int32(1), jnp.int32(0)), axis=0
    )

    jj = lax.broadcasted_iota(jnp.int32, (_THRESHOLD, w), 0)
    p = jnp.where(rank[None, :] == jj, jnp.float32(1.0), jnp.float32(0.0))

    mm = lambda a, b: jnp.dot(a, b, preferred_element_type=jnp.float32)

    ar = ar_ref[...]
    ai = ai_ref[...]
    asum = as_ref[...]
    bsel_r = mm(p, br_ref[...])
    bsel_i = mm(p, bi_ref[...])
    bsel_s = bsel_r + bsel_i

    compact = lambda m: lax.dot_general(
        m, p, (((1,), (1,)), ((), ())), preferred_element_type=jnp.float32
    )

    for c, (gr, gi) in enumerate(((g0r, g0i), (g1r, g1i))):
        gsr = compact(gr)
        gsi = compact(gi)
        m1 = mm(ar, gsr)
        m2 = mm(ai, gsi)
        m3 = mm(asum, gsr + gsi)
        tr = m1 - m2
        ti = m3 - m1 - m2
        n1 = mm(tr, bsel_r)
        n2 = mm(ti, bsel_i)
        n3 = mm(tr + ti, bsel_s)
        or_ref[0, c] = n1 - n2 + jnp.float32(0.5)
        oi_ref[0, c] = n3 - n1 - n2


@functools.partial(jax.jit, static_argnums=())
def kernel(Inp_AD_C_real, Inp_AD_C_imag):
    b, c, h, w = Inp_AD_C_real.shape
    ar, ai, asum, br, bi = _dft_mats(h, w)
    cg = c // _GROUPS

    x_spec = pl.BlockSpec((1, cg, h, w), lambda ib, ig: (ib, ig, 0, 0))
    m_spec = pl.BlockSpec((h, w), lambda ib, ig: (0, 0))
    out_r, out_i = pl.pallas_call(
        _kernel,
        grid=(b, _GROUPS),
        in_specs=[x_spec, x_spec, m_spec, m_spec, m_spec, m_spec, m_spec],
        out_specs=[x_spec, x_spec],
        out_shape=[
            jax.ShapeDtypeStruct((b, c, h, w), jnp.float32),
            jax.ShapeDtypeStruct((b, c, h, w), jnp.float32),
        ],
        compiler_params=pltpu.CompilerParams(
            dimension_semantics=("parallel", "parallel"),
        ),
    )(Inp_AD_C_real, Inp_AD_C_imag, ar, ai, asum, br, bi)
    return lax.complex(out_r, out_i)
